# baseline (speedup 1.0000x reference)
import functools

import jax
import jax.numpy as jnp
from jax import lax
from jax.experimental import pallas as pl
from jax.experimental.pallas import tpu as pltpu
from jax.experimental.pallas import tpu_sc as plsc

_B = 4096
_S = 200
_D = 64
_DW = _D // 2
_NW = 32
_TOTAL = _B * _S

_GRP = 128
_NGRP = 8
_CHUNK = _GRP * _NGRP
_ROWS_PER_W = _TOTAL // _NW
_NCHUNK = _ROWS_PER_W // _CHUNK


def _gather_kernel(idx_hbm, tok_hbm, out_hbm, idx_v, rows_v, sem):
    wid = lax.axis_index("s") * 2 + lax.axis_index("c")

    def chunk_body(c, carry):
        base = pl.multiple_of(wid * _ROWS_PER_W + c * _CHUNK, _CHUNK)

        irow = pl.multiple_of(base // _GRP, _NGRP)
        pltpu.sync_copy(idx_hbm.at[pl.ds(irow, _NGRP)], idx_v)

        copies = []
        for j in range(_NGRP):
            copies.append(
                pltpu.async_copy(
                    tok_hbm.at[idx_v.at[j]],
                    rows_v.at[pl.ds(j * _GRP, _GRP)],
                    sem,
                )
            )
        for cp in copies:
            cp.wait()

        srow = pl.multiple_of((base // _B) * _DW, _DW)
        b0 = pl.multiple_of(lax.rem(base, _B), _CHUNK)
        ocopies = []
        for w in range(_DW):
            ocopies.append(
                pltpu.async_copy(
                    rows_v.at[:, pl.ds(w, 1)],
                    out_hbm.at[srow + w, pl.ds(b0, _CHUNK), :],
                    sem,
                )
            )
        for cp in ocopies:
            cp.wait()
        return carry

    lax.fori_loop(0, _NCHUNK, chunk_body, 0)


def _add_kernel(g_ref, pos_ref, o_ref):
    t = g_ref[...].reshape(_DW, _B)
    f = pltpu.bitcast(t, jnp.bfloat16)
    s = pl.program_id(0)
    p8 = jnp.swapaxes(pos_ref[...], 0, 1)
    lane = jax.lax.broadcasted_iota(jnp.int32, (_D, 8), 1)
    p = jnp.sum(
        jnp.where(lane == lax.rem(s, 8), p8, jnp.bfloat16(0)),
        axis=1,
        keepdims=True,
    )
    o_ref[...] = (f + p).reshape(1, _D, _B)


@jax.jit
def kernel(x, token_table, pos_table):
    tok_i32 = lax.bitcast_convert_type(
        token_table.reshape(token_table.shape[0], _DW, 2), jnp.int32
    )
    idx = x.T.reshape(_TOTAL // _GRP, _GRP).astype(jnp.int32)
    pos_b = pos_table[:_S]

    mesh = plsc.VectorSubcoreMesh(core_axis_name="c", subcore_axis_name="s")
    g = pl.kernel(
        _gather_kernel,
        mesh=mesh,
        compiler_params=pltpu.CompilerParams(use_tc_tiling_on_sc=False),
        out_type=jax.ShapeDtypeStruct((_S * _DW, _B, 1), jnp.int32),
        scratch_types=[
            pltpu.VMEM((_NGRP, _GRP), jnp.int32),
            pltpu.VMEM((_CHUNK, _DW), jnp.int32),
            pltpu.SemaphoreType.DMA,
        ],
    )(idx, tok_i32)

    g3 = g.reshape(_S, _DW, _B)
    out = pl.pallas_call(
        _add_kernel,
        grid=(_S,),
        in_specs=[
            pl.BlockSpec((1, _DW, _B), lambda s: (s, 0, 0)),
            pl.BlockSpec((8, _D), lambda s: (s // 8, 0)),
        ],
        out_specs=pl.BlockSpec((1, _D, _B), lambda s: (s, 0, 0)),
        out_shape=jax.ShapeDtypeStruct((_S, _D, _B), jnp.bfloat16),
    )(g3, pos_b)

    return out.transpose(2, 0, 1)


# speedup vs baseline: 11.1686x; 11.1686x over previous
import functools

import jax
import jax.numpy as jnp
from jax import lax
from jax.experimental import pallas as pl
from jax.experimental.pallas import tpu as pltpu
from jax.experimental.pallas import tpu_sc as plsc

_B = 4096
_S = 200
_D = 64
_DW = _D // 2
_NW = 32
_TOTAL = _B * _S

_GRP = 128
_NGRP = 8
_CHUNK = _GRP * _NGRP
_ROWS_PER_W = _TOTAL // _NW
_NCHUNK = _ROWS_PER_W // _CHUNK


def _gather_kernel(idx_hbm, tok_hbm, out_hbm, idx_v, rows_v, sem):
    wid = lax.axis_index("s") * 2 + lax.axis_index("c")

    def chunk_body(c, carry):
        base = pl.multiple_of(wid * _ROWS_PER_W + c * _CHUNK, _CHUNK)

        irow = pl.multiple_of(base // _GRP, _NGRP)
        pltpu.sync_copy(idx_hbm.at[pl.ds(irow, _NGRP)], idx_v)

        copies = []
        for j in range(_NGRP):
            copies.append(
                pltpu.async_copy(
                    tok_hbm.at[idx_v.at[j]],
                    rows_v.at[pl.ds(j * _GRP, _GRP)],
                    sem,
                )
            )
        for cp in copies:
            cp.wait()

        pltpu.sync_copy(rows_v, out_hbm.at[pl.ds(base, _CHUNK)])
        return carry

    lax.fori_loop(0, _NCHUNK, chunk_body, 0)


def _add_kernel(g_ref, pos_ref, o_ref):
    g = g_ref[...].reshape(_GRP, _DW)
    t = jnp.swapaxes(g, 0, 1)
    f = pltpu.bitcast(t, jnp.bfloat16)
    s = pl.program_id(0)
    p8 = jnp.swapaxes(pos_ref[...], 0, 1)
    lane = jax.lax.broadcasted_iota(jnp.int32, (_D, 8), 1)
    p = jnp.sum(
        jnp.where(lane == lax.rem(s, 8), p8, jnp.bfloat16(0)),
        axis=1,
        keepdims=True,
    )
    o_ref[...] = (f + p).reshape(1, _D, _GRP)


@jax.jit
def kernel(x, token_table, pos_table):
    tok_i32 = lax.bitcast_convert_type(
        token_table.reshape(token_table.shape[0], _DW, 2), jnp.int32
    )
    idx = x.T.reshape(_TOTAL // _GRP, _GRP).astype(jnp.int32)
    pos_b = pos_table[:_S]

    mesh = plsc.VectorSubcoreMesh(core_axis_name="c", subcore_axis_name="s")
    g = pl.kernel(
        _gather_kernel,
        mesh=mesh,
        compiler_params=pltpu.CompilerParams(use_tc_tiling_on_sc=False),
        out_type=jax.ShapeDtypeStruct((_TOTAL, _DW), jnp.int32),
        scratch_types=[
            pltpu.VMEM((_NGRP, _GRP), jnp.int32),
            pltpu.VMEM((_CHUNK, _DW), jnp.int32),
            pltpu.SemaphoreType.DMA,
        ],
    )(idx, tok_i32)

    g3 = g.reshape(_S, _B, _DW)
    out = pl.pallas_call(
        _add_kernel,
        grid=(_S, _B // _GRP),
        in_specs=[
            pl.BlockSpec((1, _GRP, _DW), lambda s, j: (s, j, 0)),
            pl.BlockSpec((8, _D), lambda s, j: (s // 8, 0)),
        ],
        out_specs=pl.BlockSpec((1, _D, _GRP), lambda s, j: (s, 0, j)),
        out_shape=jax.ShapeDtypeStruct((_S, _D, _B), jnp.bfloat16),
    )(g3, pos_b)

    return out.transpose(2, 0, 1)


# speedup vs baseline: 42.4640x; 3.8021x over previous
import functools

import jax
import jax.numpy as jnp
from jax import lax
from jax.experimental import pallas as pl
from jax.experimental.pallas import tpu as pltpu
from jax.experimental.pallas import tpu_sc as plsc

_B = 4096
_S = 200
_D = 64
_NW = 32
_TOTAL = _B * _S

_GRP = 100
_NGRP = 16
_CHUNK = _GRP * _NGRP
_ROWS_PER_W = _TOTAL // _NW
_NCHUNK = _ROWS_PER_W // _CHUNK
_BR_PER_CHUNK = _CHUNK // _S


def _emb_kernel(idx_hbm, tok_hbm, pos_hbm, out_hbm, idx_v, rows_v, pos_v, sem):
    wid = lax.axis_index("s") * 2 + lax.axis_index("c")

    pltpu.sync_copy(pos_hbm.at[pl.ds(0, _S)], pos_v)

    def chunk_body(c, carry):
        base = pl.multiple_of(wid * _ROWS_PER_W + c * _CHUNK, _CHUNK)

        irow = pl.multiple_of(base // _GRP, _NGRP)
        pltpu.sync_copy(idx_hbm.at[pl.ds(irow, _NGRP)], idx_v)

        copies = []
        for j in range(_NGRP):
            copies.append(
                pltpu.async_copy(
                    tok_hbm.at[idx_v.at[j]],
                    rows_v.at[pl.ds(j * _GRP, _GRP)],
                    sem,
                )
            )
        for cp in copies:
            cp.wait()

        def add_body(s, carry2):
            for br in range(_BR_PER_CHUNK):
                r = br * _S
                for half in range(2):
                    t = rows_v[r + s, pl.ds(half * 32, 32)]
                    p = pos_v[s, pl.ds(half * 32, 32)]
                    rows_v[r + s, pl.ds(half * 32, 32)] = t + p
            return carry2

        lax.fori_loop(0, _S, add_body, 0)

        pltpu.sync_copy(rows_v, out_hbm.at[pl.ds(base, _CHUNK)])
        return carry

    lax.fori_loop(0, _NCHUNK, chunk_body, 0)


@jax.jit
def kernel(x, token_table, pos_table):
    idx = x.reshape(_TOTAL // _GRP, _GRP).astype(jnp.int32)

    mesh = plsc.VectorSubcoreMesh(core_axis_name="c", subcore_axis_name="s")
    out = pl.kernel(
        _emb_kernel,
        mesh=mesh,
        compiler_params=pltpu.CompilerParams(use_tc_tiling_on_sc=False),
        out_type=jax.ShapeDtypeStruct((_TOTAL, _D), jnp.bfloat16),
        scratch_types=[
            pltpu.VMEM((_NGRP, _GRP), jnp.int32),
            pltpu.VMEM((_CHUNK, _D), jnp.bfloat16),
            pltpu.VMEM((_S, _D), jnp.bfloat16),
            pltpu.SemaphoreType.DMA,
        ],
    )(idx, token_table, pos_table)

    return out.reshape(_B, _S, _D)


# speedup vs baseline: 42.6876x; 1.0053x over previous
import functools

import jax
import jax.numpy as jnp
from jax import lax
from jax.experimental import pallas as pl
from jax.experimental.pallas import tpu as pltpu
from jax.experimental.pallas import tpu_sc as plsc

_B = 4096
_S = 200
_D = 64
_NW = 32
_TOTAL = _B * _S

_GRP = 128
_NGRP = 25
_CHUNK = _GRP * _NGRP
_ROWS_PER_W = _TOTAL // _NW
_NCHUNK = _ROWS_PER_W // _CHUNK
_BR_PER_CHUNK = _CHUNK // _S


def _emb_kernel(idx_hbm, tok_hbm, pos_hbm, out_hbm, idx_v, rows_v, pos_v, sem):
    wid = lax.axis_index("s") * 2 + lax.axis_index("c")

    pltpu.sync_copy(pos_hbm.at[pl.ds(0, _S)], pos_v)

    def chunk_body(c, carry):
        base = pl.multiple_of(wid * _ROWS_PER_W + c * _CHUNK, _CHUNK)

        pltpu.sync_copy(idx_hbm.at[base // _CHUNK], idx_v)

        copies = []
        for j in range(_NGRP):
            copies.append(
                pltpu.async_copy(
                    tok_hbm.at[idx_v.at[j]],
                    rows_v.at[pl.ds(j * _GRP, _GRP)],
                    sem,
                )
            )
        for cp in copies:
            cp.wait()

        def add_body(s, carry2):
            for br in range(_BR_PER_CHUNK):
                r = br * _S
                for half in range(2):
                    t = rows_v[r + s, pl.ds(half * 32, 32)]
                    p = pos_v[s, pl.ds(half * 32, 32)]
                    rows_v[r + s, pl.ds(half * 32, 32)] = t + p
            return carry2

        lax.fori_loop(0, _S, add_body, 0)

        pltpu.sync_copy(rows_v, out_hbm.at[pl.ds(base, _CHUNK)])
        return carry

    lax.fori_loop(0, _NCHUNK, chunk_body, 0)


@jax.jit
def kernel(x, token_table, pos_table):
    idx = x.reshape(_TOTAL // _CHUNK, _NGRP, _GRP).astype(jnp.int32)

    mesh = plsc.VectorSubcoreMesh(core_axis_name="c", subcore_axis_name="s")
    out = pl.kernel(
        _emb_kernel,
        mesh=mesh,
        compiler_params=pltpu.CompilerParams(use_tc_tiling_on_sc=False),
        out_type=jax.ShapeDtypeStruct((_TOTAL, _D), jnp.bfloat16),
        scratch_types=[
            pltpu.VMEM((_NGRP, _GRP), jnp.int32),
            pltpu.VMEM((_CHUNK, _D), jnp.bfloat16),
            pltpu.VMEM((_S, _D), jnp.bfloat16),
            pltpu.SemaphoreType.DMA,
        ],
    )(idx, token_table, pos_table)

    return out.reshape(_B, _S, _D)
